# baseline (speedup 1.0000x reference)
import jax
import jax.numpy as jnp
from jax import lax
from jax.experimental import pallas as pl
from jax.experimental.pallas import tpu as pltpu

NUM_VERBS = 25
NUM_HOIS = 234
BATCH = 16384
BLOCK_B = 1024


def _gather_via_onehot(idx_ref, in_ref, out_ref):
    verb_iota = lax.broadcasted_iota(jnp.int32, (NUM_VERBS, NUM_HOIS), 0)
    onehot = (idx_ref[0][None, :] == verb_iota).astype(jnp.float32)
    out_ref[...] = jnp.dot(
        in_ref[...], onehot, preferred_element_type=jnp.float32
    )


@jax.jit
def kernel(verb_scores, hoi_to_verb):
    grid = (BATCH // BLOCK_B,)
    return pl.pallas_call(
        _gather_via_onehot,
        grid=grid,
        in_specs=[
            pl.BlockSpec((1, NUM_HOIS), lambda i: (0, 0)),
            pl.BlockSpec((BLOCK_B, NUM_VERBS), lambda i: (i, 0)),
        ],
        out_specs=pl.BlockSpec((BLOCK_B, NUM_HOIS), lambda i: (i, 0)),
        out_shape=jax.ShapeDtypeStruct((BATCH, NUM_HOIS), jnp.float32),
        compiler_params=pltpu.CompilerParams(
            dimension_semantics=("parallel",),
        ),
    )(hoi_to_verb.reshape(1, NUM_HOIS), verb_scores)


# speedup vs baseline: 1.1682x; 1.1682x over previous
import jax
import jax.numpy as jnp
from jax import lax
from jax.experimental import pallas as pl
from jax.experimental.pallas import tpu as pltpu

NUM_VERBS = 25
NUM_HOIS = 234
BATCH = 16384
BLOCK_B = 4096


def _gather_via_onehot(idx_ref, in_ref, out_ref):
    verb_iota = lax.broadcasted_iota(jnp.int32, (NUM_VERBS, NUM_HOIS), 0)
    onehot = (idx_ref[0][None, :] == verb_iota).astype(jnp.float32)
    out_ref[...] = jnp.dot(
        in_ref[...], onehot, preferred_element_type=jnp.float32
    )


@jax.jit
def kernel(verb_scores, hoi_to_verb):
    grid = (BATCH // BLOCK_B,)
    return pl.pallas_call(
        _gather_via_onehot,
        grid=grid,
        in_specs=[
            pl.BlockSpec((1, NUM_HOIS), lambda i: (0, 0)),
            pl.BlockSpec((BLOCK_B, NUM_VERBS), lambda i: (i, 0)),
        ],
        out_specs=pl.BlockSpec((BLOCK_B, NUM_HOIS), lambda i: (i, 0)),
        out_shape=jax.ShapeDtypeStruct((BATCH, NUM_HOIS), jnp.float32),
        compiler_params=pltpu.CompilerParams(
            dimension_semantics=("parallel",),
        ),
    )(hoi_to_verb.reshape(1, NUM_HOIS), verb_scores)


# speedup vs baseline: 1.2006x; 1.0277x over previous
import jax
import jax.numpy as jnp
from jax import lax
from jax.experimental import pallas as pl
from jax.experimental.pallas import tpu as pltpu

NUM_VERBS = 25
NUM_HOIS = 234
BATCH = 16384
BLOCK_B = 8192


def _gather_via_onehot(idx_ref, in_ref, out_ref):
    verb_iota = lax.broadcasted_iota(jnp.int32, (NUM_VERBS, NUM_HOIS), 0)
    onehot = (idx_ref[0][None, :] == verb_iota).astype(jnp.float32)
    out_ref[...] = jnp.dot(
        in_ref[...], onehot, preferred_element_type=jnp.float32
    )


@jax.jit
def kernel(verb_scores, hoi_to_verb):
    grid = (BATCH // BLOCK_B,)
    return pl.pallas_call(
        _gather_via_onehot,
        grid=grid,
        in_specs=[
            pl.BlockSpec((1, NUM_HOIS), lambda i: (0, 0)),
            pl.BlockSpec((BLOCK_B, NUM_VERBS), lambda i: (i, 0)),
        ],
        out_specs=pl.BlockSpec((BLOCK_B, NUM_HOIS), lambda i: (i, 0)),
        out_shape=jax.ShapeDtypeStruct((BATCH, NUM_HOIS), jnp.float32),
        compiler_params=pltpu.CompilerParams(
            dimension_semantics=("parallel",),
        ),
    )(hoi_to_verb.reshape(1, NUM_HOIS), verb_scores)
